# baseline (speedup 1.0000x reference)
import jax
import jax.numpy as jnp
from jax.experimental import pallas as pl

N_HIDDEN = 256
N_MPC = 10
N_CLASS = 100
N_CLASS_PAD = 128
N_MOTIF = N_MPC * N_CLASS
WIDTH = N_MPC * N_CLASS_PAD
N_PAD_CLASSES = N_CLASS_PAD - N_CLASS
TEMP = 0.2
EPS = 1e-4
PAD_VAL = 512.0
N_AUG = 16

BLK = 8192


def _motif_loss_kernel(z_ref, y_ref, mt_ref, msq1_ref, out_ref):
    i = pl.program_id(0)
    z = z_ref[...]
    mt = mt_ref[...]
    y = y_ref[...]
    msq1 = msq1_ref[...]

    zsq = jnp.sum(z * z, axis=1, keepdims=True)
    xp2 = jnp.dot(z.astype(jnp.bfloat16), mt,
                  preferred_element_type=jnp.float32)

    a = (zsq + msq1) + xp2
    t = a - (1.0 - EPS)
    r = a / t
    r2 = r * r
    s = r2 * r2 * r

    cs = s[:, 0:N_CLASS_PAD]
    cm = cs
    for k in range(1, N_MPC):
        chunk = s[:, k * N_CLASS_PAD:(k + 1) * N_CLASS_PAD]
        cs = cs + chunk
        cm = jnp.maximum(cm, chunk)

    cls = jax.lax.broadcasted_iota(jnp.int32, (BLK, N_CLASS_PAD), 1)
    eq = cls == y

    pos_sum = jnp.sum(jnp.where(eq, cs, 0.0), axis=1, keepdims=True)
    pos_max = jnp.max(jnp.where(eq, cm, -jnp.inf), axis=1, keepdims=True)
    total = jnp.sum(cs, axis=1, keepdims=True) - float(N_PAD_CLASSES * N_MPC)
    neg = total - pos_sum

    terms = jnp.log(pos_max) - jnp.log(neg + pos_max)
    acc = jnp.sum(terms, keepdims=True).reshape(1, 1)

    @pl.when(i == 0)
    def _init():
        out_ref[...] = jnp.zeros((1, 1), jnp.float32)

    out_ref[...] += acc


@jax.jit
def kernel(z, y, Motif_Vector):
    b = z.shape[0]

    m3 = (-2.0 * Motif_Vector).reshape(N_CLASS, N_MPC, N_HIDDEN)
    m3 = m3.transpose(2, 1, 0)
    m3 = jnp.pad(m3, ((0, 0), (0, 0), (0, N_PAD_CLASSES)),
                 constant_values=-2.0 * PAD_VAL)
    mt = m3.reshape(N_HIDDEN, WIDTH)
    msq1 = 0.25 * jnp.sum(mt * mt, axis=0, keepdims=True) + 1.0
    mt = mt.astype(jnp.bfloat16)

    y2 = y.astype(jnp.int32).reshape(b, 1)

    grid = b // BLK
    total = pl.pallas_call(
        _motif_loss_kernel,
        grid=(grid,),
        in_specs=[
            pl.BlockSpec((BLK, N_HIDDEN), lambda i: (i, 0)),
            pl.BlockSpec((BLK, 1), lambda i: (i, 0)),
            pl.BlockSpec((N_HIDDEN, WIDTH), lambda i: (0, 0)),
            pl.BlockSpec((1, WIDTH), lambda i: (0, 0)),
        ],
        out_specs=pl.BlockSpec((1, 1), lambda i: (0, 0)),
        out_shape=jax.ShapeDtypeStruct((1, 1), jnp.float32),
    )(z, y2, mt, msq1)

    return -total[0, 0] / b


# speedup vs baseline: 1.0216x; 1.0216x over previous
import jax
import jax.numpy as jnp
from jax.experimental import pallas as pl

N_HIDDEN = 256
N_MPC = 10
N_CLASS = 100
N_CLASS_PAD = 128
N_MOTIF = N_MPC * N_CLASS
WIDTH = N_MPC * N_CLASS_PAD
N_PAD_CLASSES = N_CLASS_PAD - N_CLASS
TEMP = 0.2
EPS = 1e-4
PAD_VAL = 512.0

BLK = 4096


def _motif_loss_kernel(z_ref, y_ref, mt_ref, msq1_ref, out_ref):
    i = pl.program_id(0)
    z = z_ref[...]
    mt = mt_ref[...]
    y = y_ref[...]
    msq1 = msq1_ref[...]

    zsq = jnp.sum(z * z, axis=1, keepdims=True)
    xp2 = jnp.dot(z.astype(jnp.bfloat16), mt,
                  preferred_element_type=jnp.float32)

    a = (zsq + msq1) + xp2
    t = a - (1.0 - EPS)
    r = a / t
    r2 = r * r
    s = r2 * r2 * r

    cs = s[:, 0:N_CLASS_PAD]
    cm = cs
    for k in range(1, N_MPC):
        chunk = s[:, k * N_CLASS_PAD:(k + 1) * N_CLASS_PAD]
        cs = cs + chunk
        cm = jnp.maximum(cm, chunk)

    cls = jax.lax.broadcasted_iota(jnp.int32, (BLK, N_CLASS_PAD), 1)
    eq = cls == y

    pos_sum = jnp.sum(jnp.where(eq, cs, 0.0), axis=1, keepdims=True)
    pos_max = jnp.max(jnp.where(eq, cm, -jnp.inf), axis=1, keepdims=True)
    total = jnp.sum(cs, axis=1, keepdims=True) - float(N_PAD_CLASSES * N_MPC)
    neg = total - pos_sum

    terms = jnp.log(pos_max) - jnp.log(neg + pos_max)
    acc = jnp.sum(terms, keepdims=True).reshape(1, 1)

    @pl.when(i == 0)
    def _init():
        out_ref[...] = jnp.zeros((1, 1), jnp.float32)

    out_ref[...] += acc


@jax.jit
def kernel(z, y, Motif_Vector):
    b = z.shape[0]

    m3 = (-2.0 * Motif_Vector).reshape(N_CLASS, N_MPC, N_HIDDEN)
    m3 = m3.transpose(2, 1, 0)
    m3 = jnp.pad(m3, ((0, 0), (0, 0), (0, N_PAD_CLASSES)),
                 constant_values=-2.0 * PAD_VAL)
    mt = m3.reshape(N_HIDDEN, WIDTH)
    msq1 = 0.25 * jnp.sum(mt * mt, axis=0, keepdims=True) + 1.0
    mt = mt.astype(jnp.bfloat16)

    y2 = y.astype(jnp.int32).reshape(b, 1)

    grid = b // BLK
    total = pl.pallas_call(
        _motif_loss_kernel,
        grid=(grid,),
        in_specs=[
            pl.BlockSpec((BLK, N_HIDDEN), lambda i: (i, 0)),
            pl.BlockSpec((BLK, 1), lambda i: (i, 0)),
            pl.BlockSpec((N_HIDDEN, WIDTH), lambda i: (0, 0)),
            pl.BlockSpec((1, WIDTH), lambda i: (0, 0)),
        ],
        out_specs=pl.BlockSpec((1, 1), lambda i: (0, 0)),
        out_shape=jax.ShapeDtypeStruct((1, 1), jnp.float32),
    )(z, y2, mt, msq1)

    return -total[0, 0] / b
